# baseline (speedup 1.0000x reference)
import functools

import jax
import jax.numpy as jnp
from jax import lax
from jax.experimental import pallas as pl
from jax.experimental.pallas import tpu as pltpu
from jax.experimental.pallas import tpu_sc as plsc

_B, _N, _T = 8, 64, 256
_NC, _NS = 2, 16
_NW = _NC * _NS

_SLAB_PER_W = (_B * _N) // _NW
_AC = 2


def _sc_body(tr_o, tr_p, m_o, m_p, v_o, v_p, a_o, a_p,
             o_tr_o, o_tr_p, o_m_o, o_m_p, o_v_o, o_v_p, o_a_o, o_a_p,
             tr_buf, tr_buf2, m_buf, m_buf2, v_buf, v_buf2, a_buf, a_buf2,
             sem_in, sem_out):
    wid = lax.axis_index("s") * _NC + lax.axis_index("c")
    s0 = wid * _SLAB_PER_W
    b = s0 // _N
    n0 = s0 % _N

    cin = [
        pltpu.make_async_copy(tr_o.at[b, pl.ds(n0, _SLAB_PER_W)], tr_buf, sem_in),
        pltpu.make_async_copy(tr_p.at[b, pl.ds(n0, _SLAB_PER_W)], tr_buf2, sem_in),
        pltpu.make_async_copy(m_o.at[b, pl.ds(n0, _SLAB_PER_W)], m_buf, sem_in),
        pltpu.make_async_copy(m_p.at[b, pl.ds(n0, _SLAB_PER_W)], m_buf2, sem_in),
        pltpu.make_async_copy(v_o.at[b, pl.ds(n0, _SLAB_PER_W)], v_buf, sem_in),
        pltpu.make_async_copy(v_p.at[b, pl.ds(n0, _SLAB_PER_W)], v_buf2, sem_in),
    ]
    for c in cin:
        c.start()
    for c in cin:
        c.wait()
    cout = [
        pltpu.make_async_copy(tr_buf, o_tr_o.at[b, pl.ds(n0, _SLAB_PER_W)], sem_out),
        pltpu.make_async_copy(tr_buf2, o_tr_p.at[b, pl.ds(n0, _SLAB_PER_W)], sem_out),
        pltpu.make_async_copy(m_buf, o_m_o.at[b, pl.ds(n0, _SLAB_PER_W)], sem_out),
        pltpu.make_async_copy(m_buf2, o_m_p.at[b, pl.ds(n0, _SLAB_PER_W)], sem_out),
        pltpu.make_async_copy(v_buf, o_v_o.at[b, pl.ds(n0, _SLAB_PER_W)], sem_out),
        pltpu.make_async_copy(v_buf2, o_v_p.at[b, pl.ds(n0, _SLAB_PER_W)], sem_out),
    ]
    for c in cout:
        c.start()

    nchunks = _SLAB_PER_W // _AC
    for src, dst in ((a_o, o_a_o), (a_p, o_a_p)):
        for k in range(nchunks):
            buf = a_buf if k % 2 == 0 else a_buf2
            n = n0 + k * _AC
            pltpu.make_async_copy(src.at[b, pl.ds(n, _AC)], buf, sem_in).start()
            pltpu.make_async_copy(src.at[b, pl.ds(n, _AC)], buf, sem_in).wait()
            pltpu.make_async_copy(buf, dst.at[b, pl.ds(n, _AC)], sem_out).start()
            pltpu.make_async_copy(buf, dst.at[b, pl.ds(n, _AC)], sem_out).wait()

    for c in cout:
        c.wait()


def sc_copy(tr_o, tr_p, m_o, m_p, v_ot, v_pt, a_ot, a_pt):
    mesh = plsc.VectorSubcoreMesh(core_axis_name="c", subcore_axis_name="s")
    operands = (tr_o, tr_p, m_o, m_p, v_ot, v_pt, a_ot, a_pt)
    f = pl.kernel(
        _sc_body,
        out_type=[jax.ShapeDtypeStruct(x.shape, x.dtype) for x in operands],
        mesh=mesh,
        scratch_types=[
            pltpu.VMEM((_SLAB_PER_W, 2, _T), jnp.float32),
            pltpu.VMEM((_SLAB_PER_W, 2, _T), jnp.float32),
            pltpu.VMEM((_SLAB_PER_W, _T), jnp.int32),
            pltpu.VMEM((_SLAB_PER_W, _T), jnp.int32),
            pltpu.VMEM((_SLAB_PER_W, 2, _T), jnp.float32),
            pltpu.VMEM((_SLAB_PER_W, 2, _T), jnp.float32),
            pltpu.VMEM((_AC, _N, _T), jnp.float32),
            pltpu.VMEM((_AC, _N, _T), jnp.float32),
            pltpu.SemaphoreType.DMA,
            pltpu.SemaphoreType.DMA,
        ],
    )
    return f(*operands)


def kernel(tr_o, tr_p, tr_ro, tr_rp, m_o, m_p, nl_m, inv_o, inv_p, v_o, a_o, v_p, a_p):
    v_ot = jnp.transpose(v_o, (0, 2, 3, 1))
    v_pt = jnp.transpose(v_p, (0, 2, 3, 1))
    a_ot = jnp.transpose(a_o, (0, 2, 3, 1))
    a_pt = jnp.transpose(a_p, (0, 2, 3, 1))
    outs = sc_copy(tr_o, tr_p, m_o, m_p, v_ot, v_pt, a_ot, a_pt)
    return (outs[0], outs[1], outs[2], outs[3],
            jnp.transpose(outs[4], (0, 3, 1, 2)),
            jnp.transpose(outs[5], (0, 3, 1, 2)),
            jnp.transpose(outs[6], (0, 3, 1, 2)),
            jnp.transpose(outs[7], (0, 3, 1, 2)),
            inv_o, inv_p)


# speedup vs baseline: 1.1269x; 1.1269x over previous
import jax
import jax.numpy as jnp
from jax import lax
from jax.experimental import pallas as pl
from jax.experimental.pallas import tpu as pltpu
from jax.experimental.pallas import tpu_sc as plsc

_B, _N, _T = 8, 64, 256
_NC, _NS = 2, 16
_NW = _NC * _NS
_SLAB_PER_W = (_B * _N) // _NW
_NBUF = 6
_SLACK = 3


def _sc_body(a_p, o_a_p, *rest):
    bufs = rest[:_NBUF]
    sin = rest[_NBUF:2 * _NBUF]
    sout = rest[2 * _NBUF:3 * _NBUF]
    wid = lax.axis_index("s") * _NC + lax.axis_index("c")
    s0 = wid * _SLAB_PER_W
    b = s0 // _N
    n0 = s0 % _N

    def cin(k, r):
        return pltpu.make_async_copy(a_p.at[b, pl.ds(n0 + k, 1)], bufs[r], sin[r])

    def cout(k, r):
        return pltpu.make_async_copy(bufs[r], o_a_p.at[b, pl.ds(n0 + k, 1)], sout[r])

    for j in range(_NBUF):
        cin(j, j).start()
    for k in range(_SLAB_PER_W):
        r = k % _NBUF
        cin(k, r).wait()
        cout(k, r).start()
        m = k - _SLACK
        if m >= 0 and m + _NBUF < _SLAB_PER_W:
            rr = m % _NBUF
            cout(m, rr).wait()
            cin(m + _NBUF, rr).start()
    for k in range(_SLAB_PER_W - _NBUF, _SLAB_PER_W):
        cout(k, k % _NBUF).wait()


def _sc_copy(a_pt):
    mesh = plsc.VectorSubcoreMesh(core_axis_name="c", subcore_axis_name="s")
    f = pl.kernel(
        _sc_body,
        out_type=jax.ShapeDtypeStruct(a_pt.shape, a_pt.dtype),
        mesh=mesh,
        scratch_types=(
            [pltpu.VMEM((1, _N, _T), jnp.float32)] * _NBUF
            + [pltpu.SemaphoreType.DMA] * (2 * _NBUF)
        ),
    )
    return f(a_pt)


def _copy_body(*refs):
    n = len(refs) // 2
    for i in range(n):
        refs[n + i][...] = refs[i][...]


def _tc_copy(tr_o, tr_p, m_o, m_p, v_ot, v_pt, a_ot):
    operands = (tr_o, tr_p, m_o, m_p, v_ot, v_pt, a_ot)
    tr_spec = pl.BlockSpec((1, _N, 2, _T), lambda i: (i, 0, 0, 0))
    m_spec = pl.BlockSpec((1, _N, _T), lambda i: (i, 0, 0))
    a_spec = pl.BlockSpec((1, _N, _N, _T), lambda i: (i, 0, 0, 0))
    specs = [tr_spec, tr_spec, m_spec, m_spec, tr_spec, tr_spec, a_spec]
    return pl.pallas_call(
        _copy_body,
        grid=(_B,),
        in_specs=specs,
        out_specs=specs,
        out_shape=[jax.ShapeDtypeStruct(x.shape, x.dtype) for x in operands],
    )(*operands)


def kernel(tr_o, tr_p, tr_ro, tr_rp, m_o, m_p, nl_m, inv_o, inv_p, v_o, a_o, v_p, a_p):
    v_ot = jnp.transpose(v_o, (0, 2, 3, 1))
    v_pt = jnp.transpose(v_p, (0, 2, 3, 1))
    a_ot = jnp.transpose(a_o, (0, 2, 3, 1))
    a_pt = jnp.transpose(a_p, (0, 2, 3, 1))
    o_ap = _sc_copy(a_pt)
    outs = _tc_copy(tr_o, tr_p, m_o, m_p, v_ot, v_pt, a_ot)
    return (outs[0], outs[1], outs[2], outs[3],
            jnp.transpose(outs[4], (0, 3, 1, 2)),
            jnp.transpose(outs[5], (0, 3, 1, 2)),
            jnp.transpose(outs[6], (0, 3, 1, 2)),
            jnp.transpose(o_ap, (0, 3, 1, 2)),
            inv_o, inv_p)
